# baseline (speedup 1.0000x reference)
import functools

import jax
import jax.numpy as jnp
from jax import lax
from jax.experimental import pallas as pl
from jax.experimental.pallas import tpu as pltpu
from jax.experimental.pallas import tpu_sc as plsc

N = 320000
D = 128
S = 64
NC = 2
NS = 16
NW = NC * NS
N_SC = 89600
R = N_SC // NW
L = 16
G = D // L

B1 = 400
NB1 = R // B1
CH1 = B1 // L

B2 = 80
NB2 = R // B2
CH2 = B2 // L

SG = S // 8

_mesh = plsc.VectorSubcoreMesh(
    core_axis_name="c", subcore_axis_name="s", num_cores=NC, num_subcores=NS)


def _worker_id():
    return lax.axis_index("s") * NC + lax.axis_index("c")


@functools.partial(
    pl.kernel,
    out_type=[
        jax.ShapeDtypeStruct((NW, S, D), jnp.float32),
        jax.ShapeDtypeStruct((NW, S, L), jnp.float32),
    ],
    mesh=_mesh,
    scratch_types=[
        [pltpu.VMEM((B1, D), jnp.float32)] * 2,
        [pltpu.VMEM((B1,), jnp.int32)] * 2,
        pltpu.VMEM((S, D), jnp.float32),
        pltpu.VMEM((S, L), jnp.float32),
        [pltpu.SemaphoreType.DMA] * 2,
        [pltpu.SemaphoreType.DMA] * 2,
    ],
)
def _seg_partials(feats_hbm, idx_hbm, sums_hbm, cnts_hbm,
                  rows, idxb, acc_v, cnt_v, sr, si):
    wid = _worker_id()
    base = wid * R
    zf = jnp.zeros((L,), jnp.float32)

    def zero_body(s, _):
        for j in range(G):
            acc_v[s, pl.ds(j * L, L)] = zf
        cnt_v[s, :] = zf
        return 0
    lax.fori_loop(0, S, zero_body, 0)

    def start_in(b, k):
        row0 = base + b * B1
        pltpu.async_copy(feats_hbm.at[pl.ds(row0, B1), :], rows[k], sr[k])
        pltpu.async_copy(idx_hbm.at[pl.ds(row0, B1)], idxb[k], si[k])

    def wait_in(k):
        pltpu.make_async_copy(feats_hbm.at[pl.ds(0, B1), :], rows[k], sr[k]).wait()
        pltpu.make_async_copy(idx_hbm.at[pl.ds(0, B1)], idxb[k], si[k]).wait()

    def compute(k):
        rows_v = rows[k]
        idxb_v = idxb[k]

        def chunk_body(c, _):
            iv = idxb_v[pl.ds(pl.multiple_of(c * L, L), L)]
            smin = iv[0]
            smax = iv[L - 1]
            r0 = c * L

            @pl.when(smin == smax)
            def _fast():
                for j in range(G):
                    cs = pl.ds(j * L, L)
                    sacc = rows_v[r0, cs]
                    for r in range(1, L):
                        sacc = sacc + rows_v[r0 + r, cs]
                    acc_v[smin, cs] = acc_v[smin, cs] + sacc
                cnt_v[smin, :] = cnt_v[smin, :] + jnp.full((L,), 16.0, jnp.float32)

            @pl.when(smin != smax)
            def _slow():
                for r in range(L):
                    sr_ = iv[r]
                    for j in range(G):
                        cs = pl.ds(j * L, L)
                        acc_v[sr_, cs] = acc_v[sr_, cs] + rows_v[r0 + r, cs]
                    cnt_v[sr_, :] = cnt_v[sr_, :] + jnp.full((L,), 1.0, jnp.float32)

            return 0
        lax.fori_loop(0, CH1, chunk_body, 0)

    start_in(0, 0)

    def outer(bb, _):
        for k in range(2):
            b = 2 * bb + k
            wait_in(k)
            start_in(b + 1, k ^ 1)
            compute(k)
        return 0
    lax.fori_loop(0, (NB1 - 1) // 2, outer, 0)

    wait_in(0)
    compute(0)

    pltpu.sync_copy(acc_v, sums_hbm.at[wid])
    pltpu.sync_copy(cnt_v, cnts_hbm.at[wid])


BT = 12800
NBT = N // BT
NBP0 = N_SC // BT
NBP = NBT - NBP0


def _tc_fused_body(idx_ref, sums_ref, cnts_ref, feats_ref, out_ref,
                   attn_ref, psums_ref, pcnts_ref, ones_ref):
    i = pl.program_id(0)

    @pl.when(i == 0)
    def _():
        psums_ref[...] = jnp.zeros((S, D), jnp.float32)
        pcnts_ref[...] = jnp.zeros((S, 8), jnp.float32)
        ones_ref[...] = jnp.ones((BT, 8), jnp.bfloat16)

    ids = idx_ref[0, 0, :]

    @pl.when(i < NBP)
    def _partials():
        ohb = (lax.broadcasted_iota(jnp.int32, (S, BT), 0)
               == jnp.broadcast_to(ids[None, :], (S, BT))).astype(jnp.bfloat16)
        x = feats_ref[...].astype(jnp.bfloat16)
        psums_ref[...] += jnp.dot(ohb, x, preferred_element_type=jnp.float32)
        pcnts_ref[...] += jnp.dot(ohb, ones_ref[...],
                                  preferred_element_type=jnp.float32)

    @pl.when(i == NBP)
    def _attn():
        tot = jnp.sum(sums_ref[...], axis=0) + psums_ref[...]
        cnt = jnp.sum(cnts_ref[..., :1], axis=0) + pcnts_ref[..., :1]
        attn_ref[...] = jax.nn.sigmoid(tot / jnp.maximum(cnt, 1.0))

    @pl.when(i >= NBP)
    def _apply():
        onehot = (ids[:, None] == lax.broadcasted_iota(jnp.int32, (BT, S), 1)
                  ).astype(jnp.bfloat16)
        a = attn_ref[...]
        hi = a.astype(jnp.bfloat16)
        lo = (a - hi.astype(jnp.float32)).astype(jnp.bfloat16)
        gathered = (jnp.dot(onehot, hi, preferred_element_type=jnp.float32)
                    + jnp.dot(onehot, lo, preferred_element_type=jnp.float32))
        out_ref[...] = feats_ref[...] * gathered


def _row_block(i):
    return jnp.where(i < NBP, i + NBP0, i - NBP)


_tc_fused = pl.pallas_call(
    _tc_fused_body,
    grid=(NBP + NBT,),
    in_specs=[
        pl.BlockSpec((1, 1, BT), lambda i: (_row_block(i), 0, 0)),
        pl.BlockSpec((NW, S, D), lambda i: (0, 0, 0)),
        pl.BlockSpec((NW, S, L), lambda i: (0, 0, 0)),
        pl.BlockSpec((BT, D), lambda i: (_row_block(i), 0)),
    ],
    out_specs=pl.BlockSpec((BT, D), lambda i: (jnp.where(i < NBP, 0, i - NBP), 0)),
    out_shape=jax.ShapeDtypeStruct((N, D), jnp.float32),
    scratch_shapes=[
        pltpu.VMEM((S, D), jnp.float32),
        pltpu.VMEM((S, D), jnp.float32),
        pltpu.VMEM((S, 8), jnp.float32),
        pltpu.VMEM((BT, 8), jnp.bfloat16),
    ],
)


@jax.jit
def kernel(feats, batch_idx):
    idx32 = batch_idx.astype(jnp.int32)
    idx3d = idx32.reshape(NBT, 1, BT)
    sums, cnts = _seg_partials(feats, idx32)
    return _tc_fused(idx3d, sums, cnts, feats)


# speedup vs baseline: 1.1520x; 1.1520x over previous
import functools

import jax
import jax.numpy as jnp
from jax import lax
from jax.experimental import pallas as pl
from jax.experimental.pallas import tpu as pltpu
from jax.experimental.pallas import tpu_sc as plsc

N = 320000
D = 128
S = 64
NC = 2
NS = 16
NW = NC * NS
N_SC = 89600
R = N_SC // NW
L = 16
G = D // L

B1 = 400
NB1 = R // B1
CH1 = B1 // L

B2 = 80
NB2 = R // B2
CH2 = B2 // L

SG = S // 8

_mesh = plsc.VectorSubcoreMesh(
    core_axis_name="c", subcore_axis_name="s", num_cores=NC, num_subcores=NS)


def _worker_id():
    return lax.axis_index("s") * NC + lax.axis_index("c")


@functools.partial(
    pl.kernel,
    out_type=[
        jax.ShapeDtypeStruct((NW, S, D), jnp.float32),
        jax.ShapeDtypeStruct((NW, S, L), jnp.float32),
    ],
    mesh=_mesh,
    scratch_types=[
        [pltpu.VMEM((B1, D), jnp.float32)] * 2,
        [pltpu.VMEM((B1,), jnp.int32)] * 2,
        pltpu.VMEM((S, D), jnp.float32),
        pltpu.VMEM((S, L), jnp.float32),
        [pltpu.SemaphoreType.DMA] * 2,
        [pltpu.SemaphoreType.DMA] * 2,
    ],
)
def _seg_partials(feats_hbm, idx_hbm, sums_hbm, cnts_hbm,
                  rows, idxb, acc_v, cnt_v, sr, si):
    wid = _worker_id()
    base = wid * R
    zf = jnp.zeros((L,), jnp.float32)

    def zero_body(s, _):
        for j in range(G):
            acc_v[s, pl.ds(j * L, L)] = zf
        cnt_v[s, :] = zf
        return 0
    lax.fori_loop(0, S, zero_body, 0)

    def start_in(b, k):
        row0 = base + b * B1
        pltpu.async_copy(feats_hbm.at[pl.ds(row0, B1), :], rows[k], sr[k])
        pltpu.async_copy(idx_hbm.at[pl.ds(row0, B1)], idxb[k], si[k])

    def wait_in(k):
        pltpu.make_async_copy(feats_hbm.at[pl.ds(0, B1), :], rows[k], sr[k]).wait()
        pltpu.make_async_copy(idx_hbm.at[pl.ds(0, B1)], idxb[k], si[k]).wait()

    def compute(k):
        rows_v = rows[k]
        idxb_v = idxb[k]

        def chunk_body(c, _):
            iv = idxb_v[pl.ds(pl.multiple_of(c * L, L), L)]
            smin = iv[0]
            smax = iv[L - 1]
            r0 = c * L

            @pl.when(smin == smax)
            def _fast():
                for j in range(G):
                    cs = pl.ds(j * L, L)
                    sacc = rows_v[r0, cs]
                    for r in range(1, L):
                        sacc = sacc + rows_v[r0 + r, cs]
                    acc_v[smin, cs] = acc_v[smin, cs] + sacc
                cnt_v[smin, :] = cnt_v[smin, :] + jnp.full((L,), 16.0, jnp.float32)

            @pl.when(smin != smax)
            def _slow():
                for r in range(L):
                    sr_ = iv[r]
                    for j in range(G):
                        cs = pl.ds(j * L, L)
                        acc_v[sr_, cs] = acc_v[sr_, cs] + rows_v[r0 + r, cs]
                    cnt_v[sr_, :] = cnt_v[sr_, :] + jnp.full((L,), 1.0, jnp.float32)

            return 0
        lax.fori_loop(0, CH1, chunk_body, 0)

    start_in(0, 0)

    def outer(bb, _):
        for k in range(2):
            b = 2 * bb + k
            wait_in(k)
            start_in(b + 1, k ^ 1)
            compute(k)
        return 0
    lax.fori_loop(0, (NB1 - 1) // 2, outer, 0)

    wait_in(0)
    compute(0)

    pltpu.sync_copy(acc_v, sums_hbm.at[wid])
    pltpu.sync_copy(cnt_v, cnts_hbm.at[wid])


BT = 12800
NBT = N // BT
NBP0 = N_SC // BT
NBP = NBT - NBP0
BA = 16000
NBA = N // BA


def _tc_partials_body(idx_ref, feats_ref, sums_ref, cnts_ref, ones_ref):
    @pl.when(pl.program_id(0) == 0)
    def _():
        sums_ref[...] = jnp.zeros((S, D), jnp.float32)
        cnts_ref[...] = jnp.zeros((S, 8), jnp.float32)
        ones_ref[...] = jnp.ones((BT, 8), jnp.bfloat16)

    ids = idx_ref[0, 0, :]
    ohb = (lax.broadcasted_iota(jnp.int32, (S, BT), 0)
           == jnp.broadcast_to(ids[None, :], (S, BT))).astype(jnp.bfloat16)
    x = feats_ref[...].astype(jnp.bfloat16)
    sums_ref[...] += jnp.dot(ohb, x, preferred_element_type=jnp.float32)
    cnts_ref[...] += jnp.dot(ohb, ones_ref[...],
                             preferred_element_type=jnp.float32)


_tc_partials = pl.pallas_call(
    _tc_partials_body,
    grid=(NBP,),
    in_specs=[
        pl.BlockSpec((1, 1, BT), lambda i: (i + NBP0, 0, 0)),
        pl.BlockSpec((BT, D), lambda i: (i + NBP0, 0)),
    ],
    out_specs=[
        pl.BlockSpec((S, D), lambda i: (0, 0)),
        pl.BlockSpec((S, 8), lambda i: (0, 0)),
    ],
    out_shape=[
        jax.ShapeDtypeStruct((S, D), jnp.float32),
        jax.ShapeDtypeStruct((S, 8), jnp.float32),
    ],
    scratch_shapes=[pltpu.VMEM((BT, 8), jnp.bfloat16)],
)


def _tc_apply_body(idx_ref, sums_ref, cnts_ref, tcsums_ref, tccnts_ref,
                   feats_ref, out_ref, attn_ref):
    @pl.when(pl.program_id(0) == 0)
    def _():
        tot = jnp.sum(sums_ref[...], axis=0) + tcsums_ref[...]
        cnt = (jnp.sum(cnts_ref[..., :1], axis=0)
               + tccnts_ref[..., :1])
        attn_ref[...] = jax.nn.sigmoid(tot / jnp.maximum(cnt, 1.0))

    ids = idx_ref[0, 0, :]
    onehot = (ids[:, None] == lax.broadcasted_iota(jnp.int32, (BA, S), 1)
              ).astype(jnp.bfloat16)
    a = attn_ref[...]
    hi = a.astype(jnp.bfloat16)
    lo = (a - hi.astype(jnp.float32)).astype(jnp.bfloat16)
    gathered = (jnp.dot(onehot, hi, preferred_element_type=jnp.float32)
                + jnp.dot(onehot, lo, preferred_element_type=jnp.float32))
    out_ref[...] = feats_ref[...] * gathered


_tc_apply = pl.pallas_call(
    _tc_apply_body,
    grid=(NBA,),
    in_specs=[
        pl.BlockSpec((1, 1, BA), lambda i: (i, 0, 0)),
        pl.BlockSpec((NW, S, D), lambda i: (0, 0, 0)),
        pl.BlockSpec((NW, S, L), lambda i: (0, 0, 0)),
        pl.BlockSpec((S, D), lambda i: (0, 0)),
        pl.BlockSpec((S, 8), lambda i: (0, 0)),
        pl.BlockSpec((BA, D), lambda i: (i, 0)),
    ],
    out_specs=pl.BlockSpec((BA, D), lambda i: (i, 0)),
    out_shape=jax.ShapeDtypeStruct((N, D), jnp.float32),
    scratch_shapes=[pltpu.VMEM((S, D), jnp.float32)],
)


@jax.jit
def kernel(feats, batch_idx):
    idx32 = batch_idx.astype(jnp.int32)
    idx3d = idx32.reshape(NBT, 1, BT)
    sums, cnts = _seg_partials(feats, idx32)
    tcsums, tccnts = _tc_partials(idx3d, feats)
    idx3a = idx32.reshape(NBA, 1, BA)
    return _tc_apply(idx3a, sums, cnts, tcsums, tccnts, feats)


# speedup vs baseline: 1.1528x; 1.0007x over previous
import functools

import jax
import jax.numpy as jnp
from jax import lax
from jax.experimental import pallas as pl
from jax.experimental.pallas import tpu as pltpu
from jax.experimental.pallas import tpu_sc as plsc

N = 320000
D = 128
S = 64
NC = 2
NS = 16
NW = NC * NS
N_SC = 89600
R = N_SC // NW
L = 16
G = D // L

B1 = 400
NB1 = R // B1
CH1 = B1 // L

B2 = 80
NB2 = R // B2
CH2 = B2 // L

SG = S // 8

_mesh = plsc.VectorSubcoreMesh(
    core_axis_name="c", subcore_axis_name="s", num_cores=NC, num_subcores=NS)


def _worker_id():
    return lax.axis_index("s") * NC + lax.axis_index("c")


@functools.partial(
    pl.kernel,
    out_type=[
        jax.ShapeDtypeStruct((NW, S, D), jnp.float32),
        jax.ShapeDtypeStruct((NW, S, L), jnp.float32),
    ],
    mesh=_mesh,
    scratch_types=[
        [pltpu.VMEM((B1, D), jnp.float32)] * 2,
        [pltpu.VMEM((B1,), jnp.int32)] * 2,
        pltpu.VMEM((S, D), jnp.float32),
        pltpu.VMEM((S, L), jnp.float32),
        [pltpu.SemaphoreType.DMA] * 2,
        [pltpu.SemaphoreType.DMA] * 2,
    ],
)
def _seg_partials(feats_hbm, idx_hbm, sums_hbm, cnts_hbm,
                  rows, idxb, acc_v, cnt_v, sr, si):
    wid = _worker_id()
    base = wid * R
    zf = jnp.zeros((L,), jnp.float32)

    def zero_body(s, _):
        for j in range(G):
            acc_v[s, pl.ds(j * L, L)] = zf
        cnt_v[s, :] = zf
        return 0
    lax.fori_loop(0, S, zero_body, 0)

    def start_in(b, k):
        row0 = base + b * B1
        pltpu.async_copy(feats_hbm.at[pl.ds(row0, B1), :], rows[k], sr[k])
        pltpu.async_copy(idx_hbm.at[pl.ds(row0, B1)], idxb[k], si[k])

    def wait_in(k):
        pltpu.make_async_copy(feats_hbm.at[pl.ds(0, B1), :], rows[k], sr[k]).wait()
        pltpu.make_async_copy(idx_hbm.at[pl.ds(0, B1)], idxb[k], si[k]).wait()

    def compute(k):
        rows_v = rows[k]
        idxb_v = idxb[k]

        def chunk_body(c, _):
            iv = idxb_v[pl.ds(pl.multiple_of(c * L, L), L)]
            smin = iv[0]
            smax = iv[L - 1]
            r0 = c * L

            @pl.when(smin == smax)
            def _fast():
                for j in range(G):
                    cs = pl.ds(j * L, L)
                    sacc = rows_v[r0, cs]
                    for r in range(1, L):
                        sacc = sacc + rows_v[r0 + r, cs]
                    acc_v[smin, cs] = acc_v[smin, cs] + sacc
                cnt_v[smin, :] = cnt_v[smin, :] + jnp.full((L,), 16.0, jnp.float32)

            @pl.when(smin != smax)
            def _slow():
                for r in range(L):
                    sr_ = iv[r]
                    for j in range(G):
                        cs = pl.ds(j * L, L)
                        acc_v[sr_, cs] = acc_v[sr_, cs] + rows_v[r0 + r, cs]
                    cnt_v[sr_, :] = cnt_v[sr_, :] + jnp.full((L,), 1.0, jnp.float32)

            return 0
        lax.fori_loop(0, CH1, chunk_body, 0)

    start_in(0, 0)

    def outer(bb, _):
        for k in range(2):
            b = 2 * bb + k
            wait_in(k)
            start_in(b + 1, k ^ 1)
            compute(k)
        return 0
    lax.fori_loop(0, (NB1 - 1) // 2, outer, 0)

    wait_in(0)
    compute(0)

    pltpu.sync_copy(acc_v, sums_hbm.at[wid])
    pltpu.sync_copy(cnt_v, cnts_hbm.at[wid])


BT = 12800
NBT = N // BT
NBP0 = N_SC // BT
NBP = NBT - NBP0
BA = 16000
NBA = N // BA


def _tc_partials_body(idx_ref, feats_ref, sums_ref, cnts_ref, ones_ref):
    @pl.when(pl.program_id(0) == 0)
    def _():
        sums_ref[...] = jnp.zeros((S, D), jnp.float32)
        cnts_ref[...] = jnp.zeros((S, 8), jnp.float32)
        ones_ref[...] = jnp.ones((BT, 8), jnp.bfloat16)

    ids = idx_ref[0, 0, :]
    ohb = (lax.broadcasted_iota(jnp.int32, (S, BT), 0)
           == jnp.broadcast_to(ids[None, :], (S, BT))).astype(jnp.bfloat16)
    x = feats_ref[...].astype(jnp.bfloat16)
    sums_ref[...] += jnp.dot(ohb, x, preferred_element_type=jnp.float32)
    cnts_ref[...] += jnp.dot(ohb, ones_ref[...],
                             preferred_element_type=jnp.float32)


_tc_partials = pl.pallas_call(
    _tc_partials_body,
    grid=(NBP,),
    in_specs=[
        pl.BlockSpec((1, 1, BT), lambda i: (i + NBP0, 0, 0)),
        pl.BlockSpec((BT, D), lambda i: (i + NBP0, 0)),
    ],
    out_specs=[
        pl.BlockSpec((S, D), lambda i: (0, 0)),
        pl.BlockSpec((S, 8), lambda i: (0, 0)),
    ],
    out_shape=[
        jax.ShapeDtypeStruct((S, D), jnp.float32),
        jax.ShapeDtypeStruct((S, 8), jnp.float32),
    ],
    scratch_shapes=[pltpu.VMEM((BT, 8), jnp.bfloat16)],
)


def _tc_apply_body(idx_ref, sums_ref, cnts_ref, tcsums_ref, tccnts_ref,
                   feats_ref, out_ref, attn_ref):
    @pl.when(pl.program_id(0) == 0)
    def _():
        tot = jnp.sum(sums_ref[...], axis=0) + tcsums_ref[...]
        cnt = (jnp.sum(cnts_ref[..., :1], axis=0)
               + tccnts_ref[..., :1])
        attn_ref[...] = jax.nn.sigmoid(tot / jnp.maximum(cnt, 1.0))

    ids = idx_ref[0, 0, :]
    onehot = (ids[:, None] == lax.broadcasted_iota(jnp.int32, (BA, S), 1)
              ).astype(jnp.bfloat16)
    a = attn_ref[...]
    hi = a.astype(jnp.bfloat16)
    gathered = jnp.dot(onehot, hi, preferred_element_type=jnp.float32)
    out_ref[...] = feats_ref[...] * gathered


_tc_apply = pl.pallas_call(
    _tc_apply_body,
    grid=(NBA,),
    in_specs=[
        pl.BlockSpec((1, 1, BA), lambda i: (i, 0, 0)),
        pl.BlockSpec((NW, S, D), lambda i: (0, 0, 0)),
        pl.BlockSpec((NW, S, L), lambda i: (0, 0, 0)),
        pl.BlockSpec((S, D), lambda i: (0, 0)),
        pl.BlockSpec((S, 8), lambda i: (0, 0)),
        pl.BlockSpec((BA, D), lambda i: (i, 0)),
    ],
    out_specs=pl.BlockSpec((BA, D), lambda i: (i, 0)),
    out_shape=jax.ShapeDtypeStruct((N, D), jnp.float32),
    scratch_shapes=[pltpu.VMEM((S, D), jnp.float32)],
)


@jax.jit
def kernel(feats, batch_idx):
    idx32 = batch_idx.astype(jnp.int32)
    idx3d = idx32.reshape(NBT, 1, BT)
    sums, cnts = _seg_partials(feats, idx32)
    tcsums, tccnts = _tc_partials(idx3d, feats)
    idx3a = idx32.reshape(NBA, 1, BA)
    return _tc_apply(idx3a, sums, cnts, tcsums, tccnts, feats)


# speedup vs baseline: 1.1530x; 1.0001x over previous
import functools

import jax
import jax.numpy as jnp
from jax import lax
from jax.experimental import pallas as pl
from jax.experimental.pallas import tpu as pltpu
from jax.experimental.pallas import tpu_sc as plsc

N = 320000
D = 128
S = 64
NC = 2
NS = 16
NW = NC * NS
N_SC = 76800
R = N_SC // NW
L = 16
G = D // L

B1 = 160
NB1 = R // B1
CH1 = B1 // L

B2 = 80
NB2 = R // B2
CH2 = B2 // L

SG = S // 8

_mesh = plsc.VectorSubcoreMesh(
    core_axis_name="c", subcore_axis_name="s", num_cores=NC, num_subcores=NS)


def _worker_id():
    return lax.axis_index("s") * NC + lax.axis_index("c")


@functools.partial(
    pl.kernel,
    out_type=[
        jax.ShapeDtypeStruct((NW, S, D), jnp.float32),
        jax.ShapeDtypeStruct((NW, S, L), jnp.float32),
    ],
    mesh=_mesh,
    scratch_types=[
        [pltpu.VMEM((B1, D), jnp.float32)] * 2,
        [pltpu.VMEM((B1,), jnp.int32)] * 2,
        pltpu.VMEM((S, D), jnp.float32),
        pltpu.VMEM((S, L), jnp.float32),
        [pltpu.SemaphoreType.DMA] * 2,
        [pltpu.SemaphoreType.DMA] * 2,
    ],
)
def _seg_partials(feats_hbm, idx_hbm, sums_hbm, cnts_hbm,
                  rows, idxb, acc_v, cnt_v, sr, si):
    wid = _worker_id()
    base = wid * R
    zf = jnp.zeros((L,), jnp.float32)

    def zero_body(s, _):
        for j in range(G):
            acc_v[s, pl.ds(j * L, L)] = zf
        cnt_v[s, :] = zf
        return 0
    lax.fori_loop(0, S, zero_body, 0)

    def start_in(b, k):
        row0 = base + b * B1
        pltpu.async_copy(feats_hbm.at[pl.ds(row0, B1), :], rows[k], sr[k])
        pltpu.async_copy(idx_hbm.at[pl.ds(row0, B1)], idxb[k], si[k])

    def wait_in(k):
        pltpu.make_async_copy(feats_hbm.at[pl.ds(0, B1), :], rows[k], sr[k]).wait()
        pltpu.make_async_copy(idx_hbm.at[pl.ds(0, B1)], idxb[k], si[k]).wait()

    def compute(k):
        rows_v = rows[k]
        idxb_v = idxb[k]

        def chunk_body(c, _):
            iv = idxb_v[pl.ds(pl.multiple_of(c * L, L), L)]
            smin = iv[0]
            smax = iv[L - 1]
            r0 = c * L

            @pl.when(smin == smax)
            def _fast():
                for j in range(G):
                    cs = pl.ds(j * L, L)
                    sacc = rows_v[r0, cs]
                    for r in range(1, L):
                        sacc = sacc + rows_v[r0 + r, cs]
                    acc_v[smin, cs] = acc_v[smin, cs] + sacc
                cnt_v[smin, :] = cnt_v[smin, :] + jnp.full((L,), 16.0, jnp.float32)

            @pl.when(smin != smax)
            def _slow():
                for r in range(L):
                    sr_ = iv[r]
                    for j in range(G):
                        cs = pl.ds(j * L, L)
                        acc_v[sr_, cs] = acc_v[sr_, cs] + rows_v[r0 + r, cs]
                    cnt_v[sr_, :] = cnt_v[sr_, :] + jnp.full((L,), 1.0, jnp.float32)

            return 0
        lax.fori_loop(0, CH1, chunk_body, 0)

    start_in(0, 0)

    def outer(bb, _):
        for k in range(2):
            b = 2 * bb + k
            wait_in(k)
            start_in(b + 1, k ^ 1)
            compute(k)
        return 0
    lax.fori_loop(0, (NB1 - 1) // 2, outer, 0)

    wait_in(0)
    compute(0)

    pltpu.sync_copy(acc_v, sums_hbm.at[wid])
    pltpu.sync_copy(cnt_v, cnts_hbm.at[wid])


BT = 12800
NBT = N // BT
NBP0 = N_SC // BT
NBP = NBT - NBP0
BA = 16000
NBA = N // BA


def _tc_partials_body(idx_ref, feats_ref, sums_ref, cnts_ref, ones_ref):
    @pl.when(pl.program_id(0) == 0)
    def _():
        sums_ref[...] = jnp.zeros((S, D), jnp.float32)
        cnts_ref[...] = jnp.zeros((S, 8), jnp.float32)
        ones_ref[...] = jnp.ones((BT, 8), jnp.bfloat16)

    ids = idx_ref[0, 0, :]
    ohb = (lax.broadcasted_iota(jnp.int32, (S, BT), 0)
           == jnp.broadcast_to(ids[None, :], (S, BT))).astype(jnp.bfloat16)
    x = feats_ref[...].astype(jnp.bfloat16)
    sums_ref[...] += jnp.dot(ohb, x, preferred_element_type=jnp.float32)
    cnts_ref[...] += jnp.dot(ohb, ones_ref[...],
                             preferred_element_type=jnp.float32)


_tc_partials = pl.pallas_call(
    _tc_partials_body,
    grid=(NBP,),
    in_specs=[
        pl.BlockSpec((1, 1, BT), lambda i: (i + NBP0, 0, 0)),
        pl.BlockSpec((BT, D), lambda i: (i + NBP0, 0)),
    ],
    out_specs=[
        pl.BlockSpec((S, D), lambda i: (0, 0)),
        pl.BlockSpec((S, 8), lambda i: (0, 0)),
    ],
    out_shape=[
        jax.ShapeDtypeStruct((S, D), jnp.float32),
        jax.ShapeDtypeStruct((S, 8), jnp.float32),
    ],
    scratch_shapes=[pltpu.VMEM((BT, 8), jnp.bfloat16)],
)


def _tc_apply_body(idx_ref, sums_ref, cnts_ref, tcsums_ref, tccnts_ref,
                   feats_ref, out_ref, attn_ref):
    @pl.when(pl.program_id(0) == 0)
    def _():
        tot = jnp.sum(sums_ref[...], axis=0) + tcsums_ref[...]
        cnt = (jnp.sum(cnts_ref[..., :1], axis=0)
               + tccnts_ref[..., :1])
        attn_ref[...] = jax.nn.sigmoid(tot / jnp.maximum(cnt, 1.0))

    ids = idx_ref[0, 0, :]
    onehot = (ids[:, None] == lax.broadcasted_iota(jnp.int32, (BA, S), 1)
              ).astype(jnp.bfloat16)
    a = attn_ref[...]
    hi = a.astype(jnp.bfloat16)
    lo = (a - hi.astype(jnp.float32)).astype(jnp.bfloat16)
    gathered = (jnp.dot(onehot, hi, preferred_element_type=jnp.float32)
                + jnp.dot(onehot, lo, preferred_element_type=jnp.float32))
    out_ref[...] = feats_ref[...] * gathered


_tc_apply = pl.pallas_call(
    _tc_apply_body,
    grid=(NBA,),
    in_specs=[
        pl.BlockSpec((1, 1, BA), lambda i: (i, 0, 0)),
        pl.BlockSpec((NW, S, D), lambda i: (0, 0, 0)),
        pl.BlockSpec((NW, S, L), lambda i: (0, 0, 0)),
        pl.BlockSpec((S, D), lambda i: (0, 0)),
        pl.BlockSpec((S, 8), lambda i: (0, 0)),
        pl.BlockSpec((BA, D), lambda i: (i, 0)),
    ],
    out_specs=pl.BlockSpec((BA, D), lambda i: (i, 0)),
    out_shape=jax.ShapeDtypeStruct((N, D), jnp.float32),
    scratch_shapes=[pltpu.VMEM((S, D), jnp.float32)],
)


@jax.jit
def kernel(feats, batch_idx):
    idx32 = batch_idx.astype(jnp.int32)
    idx3d = idx32.reshape(NBT, 1, BT)
    sums, cnts = _seg_partials(feats, idx32)
    tcsums, tccnts = _tc_partials(idx3d, feats)
    idx3a = idx32.reshape(NBA, 1, BA)
    return _tc_apply(idx3a, sums, cnts, tcsums, tccnts, feats)
